# initial kernel scaffold (unmeasured)
import functools

import jax
import jax.numpy as jnp
from jax import lax
from jax.experimental import pallas as pl
from jax.experimental.pallas import tpu as pltpu

N_DEV = 4
B, SQ, D = 4, 256, 1024
HQ, HKV, DH = 8, 2, 128
G = HQ // HKV
SCALE = 0.08838834764831843
R = B * SQ


def kernel(x, Wq, Wo, K_ext, V_ext):
    x2d = x.reshape(R, D)

    def body(x_ref, wq_ref, wo_ref, k_ref, v_ref, out_ref,
             o_src, s_src, o_rcv, s_rcv, acc_o, acc_m, acc_l,
             send_o_sems, recv_o_sems, send_s_sems, recv_s_sems):
        my = lax.axis_index("i")

        barrier = pltpu.get_barrier_semaphore()
        for d in range(1, N_DEV):
            pl.semaphore_signal(barrier, inc=1,
                                device_id=((my + d) % N_DEV,),
                                device_id_type=pl.DeviceIdType.MESH)
        pl.semaphore_wait(barrier, N_DEV - 1)

        q = jnp.dot(x_ref[...].astype(jnp.bfloat16),
                    wq_ref[...].astype(jnp.bfloat16),
                    preferred_element_type=jnp.float32).astype(jnp.bfloat16)

        for b in range(B):
            for g in range(HKV):
                k = k_ref[b, :, g, :].astype(jnp.bfloat16)
                v = v_ref[b, :, g, :].astype(jnp.bfloat16)
                for hh in range(G):
                    h = g * G + hh
                    qbh = q[b * SQ:(b + 1) * SQ, h * DH:(h + 1) * DH]
                    s = lax.dot_general(
                        qbh, k, (((1,), (1,)), ((), ())),
                        preferred_element_type=jnp.float32) * SCALE
                    m = jnp.max(s, axis=1, keepdims=True)
                    p = jnp.exp(s - m)
                    l = jnp.sum(p, axis=1, keepdims=True)
                    o = jnp.dot(p.astype(jnp.bfloat16), v,
                                preferred_element_type=jnp.float32)
                    o_src[b * SQ:(b + 1) * SQ, h * DH:(h + 1) * DH] = o
                    s_src[b * SQ:(b + 1) * SQ, h:h + 1] = m
                    s_src[b * SQ:(b + 1) * SQ, HQ + h:HQ + h + 1] = l

        sends = []
        for d in range(1, N_DEV):
            t = (my + d) % N_DEV
            slot = N_DEV - 1 - d
            ro = pltpu.make_async_remote_copy(
                src_ref=o_src, dst_ref=o_rcv.at[slot],
                send_sem=send_o_sems.at[slot], recv_sem=recv_o_sems.at[slot],
                device_id=(t,), device_id_type=pl.DeviceIdType.MESH)
            ro.start()
            rs = pltpu.make_async_remote_copy(
                src_ref=s_src, dst_ref=s_rcv.at[slot],
                send_sem=send_s_sems.at[slot], recv_sem=recv_s_sems.at[slot],
                device_id=(t,), device_id_type=pl.DeviceIdType.MESH)
            rs.start()
            sends += [ro, rs]

        acc_o[...] = o_src[...]
        acc_m[...] = s_src[:, 0:HQ]
        acc_l[...] = s_src[:, HQ:2 * HQ]

        for slot in (0, 2, 1):
            src_dev = (my + 1 + slot) % N_DEV
            ro = pltpu.make_async_remote_copy(
                src_ref=o_src, dst_ref=o_rcv.at[slot],
                send_sem=send_o_sems.at[slot], recv_sem=recv_o_sems.at[slot],
                device_id=(src_dev,), device_id_type=pl.DeviceIdType.MESH)
            ro.wait_recv()
            rs = pltpu.make_async_remote_copy(
                src_ref=s_src, dst_ref=s_rcv.at[slot],
                send_sem=send_s_sems.at[slot], recv_sem=recv_s_sems.at[slot],
                device_id=(src_dev,), device_id_type=pl.DeviceIdType.MESH)
            rs.wait_recv()

            rm = s_rcv[slot, :, 0:HQ]
            rl = s_rcv[slot, :, HQ:2 * HQ]
            m_old = acc_m[...]
            m_new = jnp.maximum(m_old, rm)
            a = jnp.exp(m_old - m_new)
            c = jnp.exp(rm - m_new)
            acc_m[...] = m_new
            acc_l[...] = a * acc_l[...] + c * rl
            for h in range(HQ):
                cols = slice(h * DH, (h + 1) * DH)
                acc_o[:, cols] = (a[:, h:h + 1] * acc_o[:, cols]
                                  + c[:, h:h + 1] * o_rcv[slot, :, cols])

        for r in sends:
            r.wait_send()

        inv = 1.0 / acc_l[...]
        for h in range(HQ):
            cols = slice(h * DH, (h + 1) * DH)
            acc_o[:, cols] = acc_o[:, cols] * inv[:, h:h + 1]
        out_ref[...] = jnp.dot(acc_o[...].astype(jnp.bfloat16),
                               wo_ref[...].astype(jnp.bfloat16),
                               preferred_element_type=jnp.float32)

        @functools.partial(pl.run_scoped,
                           exit_sem=pltpu.SemaphoreType.REGULAR)
        def _(exit_sem):
            for d in range(1, N_DEV):
                pl.semaphore_signal(exit_sem, inc=1,
                                    device_id=((my + d) % N_DEV,),
                                    device_id_type=pl.DeviceIdType.MESH)
            pl.semaphore_wait(exit_sem, N_DEV - 1)

    out2d = pl.pallas_call(
        body,
        out_shape=jax.ShapeDtypeStruct((R, D), jnp.float32),
        in_specs=[pl.BlockSpec(memory_space=pltpu.VMEM)] * 5,
        out_specs=pl.BlockSpec(memory_space=pltpu.VMEM),
        scratch_shapes=[
            pltpu.VMEM((R, D), jnp.float32),
            pltpu.VMEM((R, 2 * HQ), jnp.float32),
            pltpu.VMEM((N_DEV - 1, R, D), jnp.float32),
            pltpu.VMEM((N_DEV - 1, R, 2 * HQ), jnp.float32),
            pltpu.VMEM((R, D), jnp.float32),
            pltpu.VMEM((R, HQ), jnp.float32),
            pltpu.VMEM((R, HQ), jnp.float32),
            pltpu.SemaphoreType.DMA((N_DEV - 1,)),
            pltpu.SemaphoreType.DMA((N_DEV - 1,)),
            pltpu.SemaphoreType.DMA((N_DEV - 1,)),
            pltpu.SemaphoreType.DMA((N_DEV - 1,)),
        ],
        compiler_params=pltpu.CompilerParams(collective_id=0),
    )(x2d, Wq, Wo, K_ext, V_ext)
    return out2d.reshape(B, SQ, D)


# baseline (device time: 133445 ns/iter reference)
import functools

import jax
import jax.numpy as jnp
from jax import lax
from jax.experimental import pallas as pl
from jax.experimental.pallas import tpu as pltpu

N_DEV = 4
B, SQ, D = 4, 256, 1024
HQ, HKV, DH = 8, 2, 128
G = HQ // HKV
SCALE = 0.08838834764831843
R = B * SQ


def kernel(x, Wq, Wo, K_ext, V_ext):
    bf16 = jnp.bfloat16
    x2d = x.reshape(R, D).astype(bf16)

    def body(x_ref, wq_ref, wo_ref, k_ref, v_ref, out_ref,
             q_scr, o_src, s_src, o_rcv, s_rcv, acc_o, acc_m, acc_l,
             send_o_sems, recv_o_sems, send_s_sems, recv_s_sems):
        my = lax.axis_index("i")

        barrier = pltpu.get_barrier_semaphore()
        for d in range(1, N_DEV):
            pl.semaphore_signal(barrier, inc=1,
                                device_id=((my + d) % N_DEV,),
                                device_id_type=pl.DeviceIdType.MESH)
        pl.semaphore_wait(barrier, N_DEV - 1)

        q_scr[...] = jnp.dot(x_ref[...], wq_ref[...],
                             preferred_element_type=jnp.float32).astype(bf16)

        for b in range(B):
            for g in range(HKV):
                k = k_ref[b, :, g, :]
                v = v_ref[b, :, g, :]
                for hh in range(G):
                    h = g * G + hh
                    qbh = q_scr[b * SQ:(b + 1) * SQ, h * DH:(h + 1) * DH]
                    s = lax.dot_general(
                        qbh, k, (((1,), (1,)), ((), ())),
                        preferred_element_type=jnp.float32) * SCALE
                    m = jnp.max(s, axis=1, keepdims=True)
                    p = jnp.exp(s - m)
                    l = jnp.sum(p, axis=1, keepdims=True)
                    o = jnp.dot(p.astype(bf16), v,
                                preferred_element_type=jnp.float32)
                    o_src[b * SQ:(b + 1) * SQ, h * DH:(h + 1) * DH] = (
                        o.astype(bf16))
                    s_src[b * SQ:(b + 1) * SQ, h:h + 1] = m
                    s_src[b * SQ:(b + 1) * SQ, HQ + h:HQ + h + 1] = l

        sends = []
        for d in range(1, N_DEV):
            t = (my + d) % N_DEV
            slot = N_DEV - 1 - d
            ro = pltpu.make_async_remote_copy(
                src_ref=o_src, dst_ref=o_rcv.at[slot],
                send_sem=send_o_sems.at[slot], recv_sem=recv_o_sems.at[slot],
                device_id=(t,), device_id_type=pl.DeviceIdType.MESH)
            ro.start()
            rs = pltpu.make_async_remote_copy(
                src_ref=s_src, dst_ref=s_rcv.at[slot],
                send_sem=send_s_sems.at[slot], recv_sem=recv_s_sems.at[slot],
                device_id=(t,), device_id_type=pl.DeviceIdType.MESH)
            rs.start()
            sends += [ro, rs]

        acc_o[...] = o_src[...].astype(jnp.float32)
        acc_m[...] = s_src[:, 0:HQ]
        acc_l[...] = s_src[:, HQ:2 * HQ]

        for slot in (0, 2, 1):
            src_dev = (my + 1 + slot) % N_DEV
            ro = pltpu.make_async_remote_copy(
                src_ref=o_src, dst_ref=o_rcv.at[slot],
                send_sem=send_o_sems.at[slot], recv_sem=recv_o_sems.at[slot],
                device_id=(src_dev,), device_id_type=pl.DeviceIdType.MESH)
            ro.wait_recv()
            rs = pltpu.make_async_remote_copy(
                src_ref=s_src, dst_ref=s_rcv.at[slot],
                send_sem=send_s_sems.at[slot], recv_sem=recv_s_sems.at[slot],
                device_id=(src_dev,), device_id_type=pl.DeviceIdType.MESH)
            rs.wait_recv()

            rm = s_rcv[slot, :, 0:HQ]
            rl = s_rcv[slot, :, HQ:2 * HQ]
            m_old = acc_m[...]
            m_new = jnp.maximum(m_old, rm)
            a = jnp.exp(m_old - m_new)
            c = jnp.exp(rm - m_new)
            acc_m[...] = m_new
            acc_l[...] = a * acc_l[...] + c * rl
            for h in range(HQ):
                cols = slice(h * DH, (h + 1) * DH)
                acc_o[:, cols] = (
                    a[:, h:h + 1] * acc_o[:, cols]
                    + c[:, h:h + 1] * o_rcv[slot, :, cols].astype(jnp.float32))

        for r in sends:
            r.wait_send()

        inv = 1.0 / acc_l[...]
        for h in range(HQ):
            cols = slice(h * DH, (h + 1) * DH)
            acc_o[:, cols] = acc_o[:, cols] * inv[:, h:h + 1]
        out_ref[...] = jnp.dot(acc_o[...].astype(bf16), wo_ref[...],
                               preferred_element_type=jnp.float32)

        @functools.partial(pl.run_scoped,
                           exit_sem=pltpu.SemaphoreType.REGULAR)
        def _(exit_sem):
            for d in range(1, N_DEV):
                pl.semaphore_signal(exit_sem, inc=1,
                                    device_id=((my + d) % N_DEV,),
                                    device_id_type=pl.DeviceIdType.MESH)
            pl.semaphore_wait(exit_sem, N_DEV - 1)

    out2d = pl.pallas_call(
        body,
        out_shape=jax.ShapeDtypeStruct((R, D), jnp.float32),
        in_specs=[pl.BlockSpec(memory_space=pltpu.VMEM)] * 5,
        out_specs=pl.BlockSpec(memory_space=pltpu.VMEM),
        scratch_shapes=[
            pltpu.VMEM((R, D), bf16),
            pltpu.VMEM((R, D), bf16),
            pltpu.VMEM((R, 2 * HQ), jnp.float32),
            pltpu.VMEM((N_DEV - 1, R, D), bf16),
            pltpu.VMEM((N_DEV - 1, R, 2 * HQ), jnp.float32),
            pltpu.VMEM((R, D), jnp.float32),
            pltpu.VMEM((R, HQ), jnp.float32),
            pltpu.VMEM((R, HQ), jnp.float32),
            pltpu.SemaphoreType.DMA((N_DEV - 1,)),
            pltpu.SemaphoreType.DMA((N_DEV - 1,)),
            pltpu.SemaphoreType.DMA((N_DEV - 1,)),
            pltpu.SemaphoreType.DMA((N_DEV - 1,)),
        ],
        compiler_params=pltpu.CompilerParams(
            collective_id=0, vmem_limit_bytes=100 * 1024 * 1024),
    )(x2d, Wq.astype(bf16), Wo.astype(bf16),
      K_ext.astype(bf16), V_ext.astype(bf16))
    return out2d.reshape(B, SQ, D)


# device time: 107183 ns/iter; 1.2450x vs baseline; 1.2450x over previous
import functools

import jax
import jax.numpy as jnp
from jax import lax
from jax.experimental import pallas as pl
from jax.experimental.pallas import tpu as pltpu

N_DEV = 4
B, SQ, D = 4, 256, 1024
HQ, HKV, DH = 8, 2, 128
G = HQ // HKV
SCALE = 0.08838834764831843
R = B * SQ


def kernel(x, Wq, Wo, K_ext, V_ext):
    bf16 = jnp.bfloat16
    x2d = x.reshape(R, D).astype(bf16)

    def body(x_ref, wq_ref, wo_ref, k_ref, v_ref, out_ref,
             o_src, s_src, o_rcv, s_rcv, acc_o, acc_m, acc_l,
             send_o_sems, recv_o_sems, send_s_sems, recv_s_sems):
        my = lax.axis_index("i")

        barrier = pltpu.get_barrier_semaphore()
        for d in range(1, N_DEV):
            pl.semaphore_signal(barrier, inc=1,
                                device_id=((my + d) % N_DEV,),
                                device_id_type=pl.DeviceIdType.MESH)
        pl.semaphore_wait(barrier, N_DEV - 1)

        def o_chunk_rdma(slot, b, dev):
            rows = pl.ds(b * SQ, SQ)
            return pltpu.make_async_remote_copy(
                src_ref=o_src.at[rows, :],
                dst_ref=o_rcv.at[slot, rows, :],
                send_sem=send_o_sems.at[slot, b],
                recv_sem=recv_o_sems.at[slot, b],
                device_id=(dev,), device_id_type=pl.DeviceIdType.MESH)

        def s_rdma(slot, dev):
            return pltpu.make_async_remote_copy(
                src_ref=s_src, dst_ref=s_rcv.at[slot],
                send_sem=send_s_sems.at[slot], recv_sem=recv_s_sems.at[slot],
                device_id=(dev,), device_id_type=pl.DeviceIdType.MESH)

        sends = []
        for b in range(B):
            qb = jnp.dot(x_ref[b * SQ:(b + 1) * SQ, :], wq_ref[...],
                         preferred_element_type=jnp.float32).astype(bf16)
            for g in range(HKV):
                k = k_ref[b, :, g, :]
                v = v_ref[b, :, g, :]
                for hh in range(G):
                    h = g * G + hh
                    s = lax.dot_general(
                        qb[:, h * DH:(h + 1) * DH], k,
                        (((1,), (1,)), ((), ())),
                        preferred_element_type=jnp.float32) * SCALE
                    m = jnp.max(s, axis=1, keepdims=True)
                    p = jnp.exp(s - m)
                    l = jnp.sum(p, axis=1, keepdims=True)
                    o = jnp.dot(p.astype(bf16), v,
                                preferred_element_type=jnp.float32)
                    o_src[b * SQ:(b + 1) * SQ, h * DH:(h + 1) * DH] = (
                        o.astype(bf16))
                    s_src[b * SQ:(b + 1) * SQ, h:h + 1] = m
                    s_src[b * SQ:(b + 1) * SQ, HQ + h:HQ + h + 1] = l
            for d in range(1, N_DEV):
                r = o_chunk_rdma(N_DEV - 1 - d, b, (my + d) % N_DEV)
                r.start()
                sends.append(r)

        for d in range(1, N_DEV):
            r = s_rdma(N_DEV - 1 - d, (my + d) % N_DEV)
            r.start()
            sends.append(r)

        acc_o[...] = o_src[...].astype(jnp.float32)
        acc_m[...] = s_src[:, 0:HQ]
        acc_l[...] = s_src[:, HQ:2 * HQ]

        for slot in (0, 2, 1):
            src_dev = (my + 1 + slot) % N_DEV
            for b in range(B):
                o_chunk_rdma(slot, b, src_dev).wait_recv()
            s_rdma(slot, src_dev).wait_recv()

            rm = s_rcv[slot, :, 0:HQ]
            rl = s_rcv[slot, :, HQ:2 * HQ]
            m_old = acc_m[...]
            m_new = jnp.maximum(m_old, rm)
            a = jnp.exp(m_old - m_new)
            c = jnp.exp(rm - m_new)
            acc_m[...] = m_new
            acc_l[...] = a * acc_l[...] + c * rl
            for h in range(HQ):
                cols = slice(h * DH, (h + 1) * DH)
                acc_o[:, cols] = (
                    a[:, h:h + 1] * acc_o[:, cols]
                    + c[:, h:h + 1] * o_rcv[slot, :, cols].astype(jnp.float32))

        for r in sends:
            r.wait_send()

        inv = 1.0 / acc_l[...]
        for h in range(HQ):
            cols = slice(h * DH, (h + 1) * DH)
            acc_o[:, cols] = acc_o[:, cols] * inv[:, h:h + 1]
        out_ref[...] = jnp.dot(acc_o[...].astype(bf16), wo_ref[...],
                               preferred_element_type=jnp.float32)

        @functools.partial(pl.run_scoped,
                           exit_sem=pltpu.SemaphoreType.REGULAR)
        def _(exit_sem):
            for d in range(1, N_DEV):
                pl.semaphore_signal(exit_sem, inc=1,
                                    device_id=((my + d) % N_DEV,),
                                    device_id_type=pl.DeviceIdType.MESH)
            pl.semaphore_wait(exit_sem, N_DEV - 1)

    out2d = pl.pallas_call(
        body,
        out_shape=jax.ShapeDtypeStruct((R, D), jnp.float32),
        in_specs=[pl.BlockSpec(memory_space=pltpu.VMEM)] * 5,
        out_specs=pl.BlockSpec(memory_space=pltpu.VMEM),
        scratch_shapes=[
            pltpu.VMEM((R, D), bf16),
            pltpu.VMEM((R, 2 * HQ), jnp.float32),
            pltpu.VMEM((N_DEV - 1, R, D), bf16),
            pltpu.VMEM((N_DEV - 1, R, 2 * HQ), jnp.float32),
            pltpu.VMEM((R, D), jnp.float32),
            pltpu.VMEM((R, HQ), jnp.float32),
            pltpu.VMEM((R, HQ), jnp.float32),
            pltpu.SemaphoreType.DMA((N_DEV - 1, B)),
            pltpu.SemaphoreType.DMA((N_DEV - 1, B)),
            pltpu.SemaphoreType.DMA((N_DEV - 1,)),
            pltpu.SemaphoreType.DMA((N_DEV - 1,)),
        ],
        compiler_params=pltpu.CompilerParams(
            collective_id=0, vmem_limit_bytes=100 * 1024 * 1024),
    )(x2d, Wq.astype(bf16), Wo.astype(bf16),
      K_ext.astype(bf16), V_ext.astype(bf16))
    return out2d.reshape(B, SQ, D)


# device time: 82617 ns/iter; 1.6152x vs baseline; 1.2973x over previous
import functools

import jax
import jax.numpy as jnp
from jax import lax
from jax.experimental import pallas as pl
from jax.experimental.pallas import tpu as pltpu

N_DEV = 4
B, SQ, D = 4, 256, 1024
HQ, HKV, DH = 8, 2, 128
G = HQ // HKV
SCALE = 0.08838834764831843
R = B * SQ


def kernel(x, Wq, Wo, K_ext, V_ext):
    bf16 = jnp.bfloat16
    f32 = jnp.float32
    my_idx = lax.axis_index("i")

    xp = jnp.roll(x, -my_idx, axis=0).reshape(R, D).astype(bf16)
    Kp = jnp.roll(K_ext, -my_idx, axis=0).astype(bf16)
    Vp = jnp.roll(V_ext, -my_idx, axis=0).astype(bf16)

    def body(x_ref, wq_ref, wo_ref, k_ref, v_ref, out_ref,
             po_src, ps_src, o_own, s_own, po_rcv, ps_rcv, og_src, og_rcv,
             so_sems, ro_sems, ss_sems, rs_sems, sg_sems, rg_sems):
        my = lax.axis_index("i")

        barrier = pltpu.get_barrier_semaphore()
        for d in range(1, N_DEV):
            pl.semaphore_signal(barrier, inc=1,
                                device_id=((my + d) % N_DEV,),
                                device_id_type=pl.DeviceIdType.MESH)
        pl.semaphore_wait(barrier, N_DEV - 1)

        sends = []
        for db in [1, 2, 3, 0]:
            if db == 0:
                o_dst, s_dst = o_own, s_own
            else:
                o_dst, s_dst = po_src.at[db - 1], ps_src.at[db - 1]
            qb = jnp.dot(x_ref[db * SQ:(db + 1) * SQ, :], wq_ref[...],
                         preferred_element_type=f32).astype(bf16)
            for g in range(HKV):
                k = k_ref[db, :, g, :]
                v = v_ref[db, :, g, :]
                for hh in range(G):
                    h = g * G + hh
                    s = lax.dot_general(
                        qb[:, h * DH:(h + 1) * DH], k,
                        (((1,), (1,)), ((), ())),
                        preferred_element_type=f32) * SCALE
                    m = jnp.max(s, axis=1, keepdims=True)
                    p = jnp.exp(s - m)
                    l = jnp.sum(p, axis=1, keepdims=True)
                    o = jnp.dot(p.astype(bf16), v,
                                preferred_element_type=f32)
                    o_dst[:, h * DH:(h + 1) * DH] = o.astype(bf16)
                    s_dst[:, h:h + 1] = m
                    s_dst[:, HQ + h:HQ + h + 1] = l
            if db != 0:
                slot = 3 - db
                ro = pltpu.make_async_remote_copy(
                    src_ref=po_src.at[db - 1], dst_ref=po_rcv.at[slot],
                    send_sem=so_sems.at[db - 1], recv_sem=ro_sems.at[slot],
                    device_id=((my + db) % N_DEV,),
                    device_id_type=pl.DeviceIdType.MESH)
                ro.start()
                rs = pltpu.make_async_remote_copy(
                    src_ref=ps_src.at[db - 1], dst_ref=ps_rcv.at[slot],
                    send_sem=ss_sems.at[db - 1], recv_sem=rs_sems.at[slot],
                    device_id=((my + db) % N_DEV,),
                    device_id_type=pl.DeviceIdType.MESH)
                rs.start()
                sends += [ro, rs]

        acc_o_v = o_own[...].astype(f32)
        acc_m = s_own[:, 0:HQ]
        acc_l = s_own[:, HQ:2 * HQ]

        for slot in (2, 1, 0):
            src_dev = (my + 1 + slot) % N_DEV
            pltpu.make_async_remote_copy(
                src_ref=po_src.at[0], dst_ref=po_rcv.at[slot],
                send_sem=so_sems.at[0], recv_sem=ro_sems.at[slot],
                device_id=(src_dev,),
                device_id_type=pl.DeviceIdType.MESH).wait_recv()
            pltpu.make_async_remote_copy(
                src_ref=ps_src.at[0], dst_ref=ps_rcv.at[slot],
                send_sem=ss_sems.at[0], recv_sem=rs_sems.at[slot],
                device_id=(src_dev,),
                device_id_type=pl.DeviceIdType.MESH).wait_recv()

            rm = ps_rcv[slot, :, 0:HQ]
            rl = ps_rcv[slot, :, HQ:2 * HQ]
            m_new = jnp.maximum(acc_m, rm)
            a = jnp.exp(acc_m - m_new)
            c = jnp.exp(rm - m_new)
            acc_m = m_new
            acc_l = a * acc_l + c * rl
            parts = []
            for h in range(HQ):
                cols = slice(h * DH, (h + 1) * DH)
                parts.append(a[:, h:h + 1] * acc_o_v[:, cols]
                             + c[:, h:h + 1]
                             * po_rcv[slot, :, cols].astype(f32))
            acc_o_v = jnp.concatenate(parts, axis=1)

        inv = 1.0 / acc_l
        parts = []
        for h in range(HQ):
            parts.append(acc_o_v[:, h * DH:(h + 1) * DH] * inv[:, h:h + 1])
        blk = jnp.dot(jnp.concatenate(parts, axis=1).astype(bf16),
                      wo_ref[...], preferred_element_type=f32)
        og_src[...] = blk.astype(bf16)
        out_ref[0:SQ, :] = blk

        for d in range(1, N_DEV):
            r = pltpu.make_async_remote_copy(
                src_ref=og_src, dst_ref=og_rcv.at[3 - d],
                send_sem=sg_sems.at[d - 1], recv_sem=rg_sems.at[3 - d],
                device_id=((my + d) % N_DEV,),
                device_id_type=pl.DeviceIdType.MESH)
            r.start()
            sends.append(r)
        for slot in (2, 1, 0):
            src_dev = (my + 1 + slot) % N_DEV
            pltpu.make_async_remote_copy(
                src_ref=og_src, dst_ref=og_rcv.at[slot],
                send_sem=sg_sems.at[0], recv_sem=rg_sems.at[slot],
                device_id=(src_dev,),
                device_id_type=pl.DeviceIdType.MESH).wait_recv()
            pos = slot + 1
            out_ref[pos * SQ:(pos + 1) * SQ, :] = (
                og_rcv[slot, :, :].astype(f32))
        for r in sends:
            r.wait_send()

        @functools.partial(pl.run_scoped,
                           exit_sem=pltpu.SemaphoreType.REGULAR)
        def _(exit_sem):
            for d in range(1, N_DEV):
                pl.semaphore_signal(exit_sem, inc=1,
                                    device_id=((my + d) % N_DEV,),
                                    device_id_type=pl.DeviceIdType.MESH)
            pl.semaphore_wait(exit_sem, N_DEV - 1)

    out2d = pl.pallas_call(
        body,
        out_shape=jax.ShapeDtypeStruct((R, D), f32),
        in_specs=[pl.BlockSpec(memory_space=pltpu.VMEM)] * 5,
        out_specs=pl.BlockSpec(memory_space=pltpu.VMEM),
        scratch_shapes=[
            pltpu.VMEM((N_DEV - 1, SQ, D), bf16),
            pltpu.VMEM((N_DEV - 1, SQ, 2 * HQ), f32),
            pltpu.VMEM((SQ, D), bf16),
            pltpu.VMEM((SQ, 2 * HQ), f32),
            pltpu.VMEM((N_DEV - 1, SQ, D), bf16),
            pltpu.VMEM((N_DEV - 1, SQ, 2 * HQ), f32),
            pltpu.VMEM((SQ, D), bf16),
            pltpu.VMEM((N_DEV - 1, SQ, D), bf16),
            pltpu.SemaphoreType.DMA((N_DEV - 1,)),
            pltpu.SemaphoreType.DMA((N_DEV - 1,)),
            pltpu.SemaphoreType.DMA((N_DEV - 1,)),
            pltpu.SemaphoreType.DMA((N_DEV - 1,)),
            pltpu.SemaphoreType.DMA((N_DEV - 1,)),
            pltpu.SemaphoreType.DMA((N_DEV - 1,)),
        ],
        compiler_params=pltpu.CompilerParams(
            collective_id=0, vmem_limit_bytes=100 * 1024 * 1024),
    )(xp, Wq.astype(bf16), Wo.astype(bf16), Kp, Vp)

    return jnp.roll(out2d.reshape(B, SQ, D), my_idx, axis=0)


# device time: 64042 ns/iter; 2.0837x vs baseline; 1.2900x over previous
import functools

import jax
import jax.numpy as jnp
from jax import lax
from jax.experimental import pallas as pl
from jax.experimental.pallas import tpu as pltpu

N_DEV = 4
B, SQ, D = 4, 256, 1024
HQ, HKV, DH = 8, 2, 128
G = HQ // HKV
SCALE = 0.08838834764831843
R = B * SQ


def kernel(x, Wq, Wo, K_ext, V_ext):
    bf16 = jnp.bfloat16
    f32 = jnp.float32
    x2d = x.reshape(R, D)

    def body(x_ref, wq_ref, wo_ref, k_ref, v_ref, out_ref,
             po_src, ps_src, o_own, s_own, po_rcv, ps_rcv, og_src, og_rcv,
             so_sems, ro_sems, ss_sems, rs_sems, sg_sems, rg_sems):
        my = lax.axis_index("i")

        barrier = pltpu.get_barrier_semaphore()
        for d in range(1, N_DEV):
            pl.semaphore_signal(barrier, inc=1,
                                device_id=((my + d) % N_DEV,),
                                device_id_type=pl.DeviceIdType.MESH)
        pl.semaphore_wait(barrier, N_DEV - 1)

        wq_b = wq_ref[...].astype(bf16)

        sends = []
        for db in [1, 2, 3, 0]:
            bb = (my + db) % N_DEV
            rows = pl.ds(bb * SQ, SQ)
            if db == 0:
                o_dst, s_dst = o_own, s_own
            else:
                o_dst, s_dst = po_src.at[db - 1], ps_src.at[db - 1]
            qb = jnp.dot(x_ref[rows, :].astype(bf16), wq_b,
                         preferred_element_type=f32).astype(bf16)
            for g in range(HKV):
                k = k_ref[bb, :, g, :].astype(bf16)
                v = v_ref[bb, :, g, :].astype(bf16)
                for hh in range(G):
                    h = g * G + hh
                    s = lax.dot_general(
                        qb[:, h * DH:(h + 1) * DH], k,
                        (((1,), (1,)), ((), ())),
                        preferred_element_type=f32) * SCALE
                    m = jnp.max(s, axis=1, keepdims=True)
                    p = jnp.exp(s - m)
                    l = jnp.sum(p, axis=1, keepdims=True)
                    o = jnp.dot(p.astype(bf16), v,
                                preferred_element_type=f32)
                    o_dst[:, h * DH:(h + 1) * DH] = o.astype(bf16)
                    s_dst[:, h:h + 1] = m
                    s_dst[:, HQ + h:HQ + h + 1] = l
            if db != 0:
                slot = 3 - db
                ro = pltpu.make_async_remote_copy(
                    src_ref=po_src.at[db - 1], dst_ref=po_rcv.at[slot],
                    send_sem=so_sems.at[db - 1], recv_sem=ro_sems.at[slot],
                    device_id=(bb,),
                    device_id_type=pl.DeviceIdType.MESH)
                ro.start()
                rs = pltpu.make_async_remote_copy(
                    src_ref=ps_src.at[db - 1], dst_ref=ps_rcv.at[slot],
                    send_sem=ss_sems.at[db - 1], recv_sem=rs_sems.at[slot],
                    device_id=(bb,),
                    device_id_type=pl.DeviceIdType.MESH)
                rs.start()
                sends += [ro, rs]

        acc_o_v = o_own[...].astype(f32)
        acc_m = s_own[:, 0:HQ]
        acc_l = s_own[:, HQ:2 * HQ]

        for slot in (2, 1, 0):
            src_dev = (my + 1 + slot) % N_DEV
            pltpu.make_async_remote_copy(
                src_ref=po_src.at[0], dst_ref=po_rcv.at[slot],
                send_sem=so_sems.at[0], recv_sem=ro_sems.at[slot],
                device_id=(src_dev,),
                device_id_type=pl.DeviceIdType.MESH).wait_recv()
            pltpu.make_async_remote_copy(
                src_ref=ps_src.at[0], dst_ref=ps_rcv.at[slot],
                send_sem=ss_sems.at[0], recv_sem=rs_sems.at[slot],
                device_id=(src_dev,),
                device_id_type=pl.DeviceIdType.MESH).wait_recv()

            rm = ps_rcv[slot, :, 0:HQ]
            rl = ps_rcv[slot, :, HQ:2 * HQ]
            m_new = jnp.maximum(acc_m, rm)
            a = jnp.exp(acc_m - m_new)
            c = jnp.exp(rm - m_new)
            acc_m = m_new
            acc_l = a * acc_l + c * rl
            parts = []
            for h in range(HQ):
                cols = slice(h * DH, (h + 1) * DH)
                parts.append(a[:, h:h + 1] * acc_o_v[:, cols]
                             + c[:, h:h + 1]
                             * po_rcv[slot, :, cols].astype(f32))
            acc_o_v = jnp.concatenate(parts, axis=1)

        inv = 1.0 / acc_l
        parts = []
        for h in range(HQ):
            parts.append(acc_o_v[:, h * DH:(h + 1) * DH] * inv[:, h:h + 1])
        blk = jnp.dot(jnp.concatenate(parts, axis=1).astype(bf16),
                      wo_ref[...].astype(bf16), preferred_element_type=f32)
        og_src[...] = blk.astype(bf16)
        out_ref[pl.ds(my * SQ, SQ), :] = blk

        for d in range(1, N_DEV):
            r = pltpu.make_async_remote_copy(
                src_ref=og_src, dst_ref=og_rcv.at[3 - d],
                send_sem=sg_sems.at[d - 1], recv_sem=rg_sems.at[3 - d],
                device_id=((my + d) % N_DEV,),
                device_id_type=pl.DeviceIdType.MESH)
            r.start()
            sends.append(r)
        for slot in (2, 1, 0):
            src_dev = (my + 1 + slot) % N_DEV
            pltpu.make_async_remote_copy(
                src_ref=og_src, dst_ref=og_rcv.at[slot],
                send_sem=sg_sems.at[0], recv_sem=rg_sems.at[slot],
                device_id=(src_dev,),
                device_id_type=pl.DeviceIdType.MESH).wait_recv()
            out_ref[pl.ds(src_dev * SQ, SQ), :] = (
                og_rcv[slot, :, :].astype(f32))
        for r in sends:
            r.wait_send()

        @functools.partial(pl.run_scoped,
                           exit_sem=pltpu.SemaphoreType.REGULAR)
        def _(exit_sem):
            for d in range(1, N_DEV):
                pl.semaphore_signal(exit_sem, inc=1,
                                    device_id=((my + d) % N_DEV,),
                                    device_id_type=pl.DeviceIdType.MESH)
            pl.semaphore_wait(exit_sem, N_DEV - 1)

    out2d = pl.pallas_call(
        body,
        out_shape=jax.ShapeDtypeStruct((R, D), f32),
        in_specs=[pl.BlockSpec(memory_space=pltpu.VMEM)] * 5,
        out_specs=pl.BlockSpec(memory_space=pltpu.VMEM),
        scratch_shapes=[
            pltpu.VMEM((N_DEV - 1, SQ, D), bf16),
            pltpu.VMEM((N_DEV - 1, SQ, 2 * HQ), f32),
            pltpu.VMEM((SQ, D), bf16),
            pltpu.VMEM((SQ, 2 * HQ), f32),
            pltpu.VMEM((N_DEV - 1, SQ, D), bf16),
            pltpu.VMEM((N_DEV - 1, SQ, 2 * HQ), f32),
            pltpu.VMEM((SQ, D), bf16),
            pltpu.VMEM((N_DEV - 1, SQ, D), bf16),
            pltpu.SemaphoreType.DMA((N_DEV - 1,)),
            pltpu.SemaphoreType.DMA((N_DEV - 1,)),
            pltpu.SemaphoreType.DMA((N_DEV - 1,)),
            pltpu.SemaphoreType.DMA((N_DEV - 1,)),
            pltpu.SemaphoreType.DMA((N_DEV - 1,)),
            pltpu.SemaphoreType.DMA((N_DEV - 1,)),
        ],
        compiler_params=pltpu.CompilerParams(
            collective_id=0, vmem_limit_bytes=100 * 1024 * 1024),
    )(x2d, Wq, Wo, K_ext, V_ext)
    return out2d.reshape(B, SQ, D)
